# Initial kernel scaffold; baseline (speedup 1.0000x reference)
import jax
import jax.numpy as jnp
from jax.experimental import pallas as pl


def kernel(x, gamma1, beta1, w1, b1, gamma2, beta2, w2, b2):
    raise NotImplementedError("write your pallas kernel here")


# speedup vs baseline: 1.3241x; 1.3241x over previous
import functools

import jax
import jax.numpy as jnp
from jax.experimental import pallas as pl
from jax.experimental.pallas import tpu as pltpu

_VMEM_LIMIT = 48 * 1024 * 1024
_EPS = 1e-5


def _stats_kernel(x_ref, sum_ref, sq_ref):
    @pl.when(pl.program_id(0) == 0)
    def _init():
        sum_ref[...] = jnp.zeros_like(sum_ref)
        sq_ref[...] = jnp.zeros_like(sq_ref)

    xv = x_ref[...].astype(jnp.float32)
    sum_ref[...] += jnp.sum(xv, axis=0, keepdims=True)
    sq_ref[...] += jnp.sum(xv * xv, axis=0, keepdims=True)


def _channel_stats(x_f, C):
    L = x_f.shape[-1]
    x2d = x_f.reshape(-1, L)
    rows = x2d.shape[0]
    tm = min(rows, 2048)
    while rows % tm:
        tm //= 2

    sums, sq = pl.pallas_call(
        _stats_kernel,
        out_shape=(jax.ShapeDtypeStruct((1, L), jnp.float32),
                   jax.ShapeDtypeStruct((1, L), jnp.float32)),
        grid=(rows // tm,),
        in_specs=[pl.BlockSpec((tm, L), lambda i: (i, 0))],
        out_specs=(pl.BlockSpec((1, L), lambda i: (0, 0)),
                   pl.BlockSpec((1, L), lambda i: (0, 0))),
        compiler_params=pltpu.CompilerParams(
            dimension_semantics=("arbitrary",),
            vmem_limit_bytes=_VMEM_LIMIT),
    )(x2d)

    ch_sum = sums.reshape(L // C, C).sum(axis=0)
    ch_sq = sq.reshape(L // C, C).sum(axis=0)
    return ch_sum, ch_sq


def _bn_coeffs(ch_sum, ch_sq, count, gamma, beta):
    mean = ch_sum / count
    var = ch_sq / count - mean * mean
    scale = gamma.astype(jnp.float32) * jax.lax.rsqrt(var + _EPS)
    shift = beta.astype(jnp.float32) - mean * scale
    return scale, shift


def _expand_weight(w_dhwio, Wt):
    KD, KH, KW, Ci, Co = w_dhwio.shape
    WQ = Wt + 2
    sel = (jnp.arange(WQ)[None, :, None] ==
           (jnp.arange(Wt)[None, None, :] + jnp.arange(KW)[:, None, None]))
    sel = sel.astype(w_dhwio.dtype)
    wexp = jnp.einsum("kxw,dhkio->dhxiwo", sel, w_dhwio)
    return wexp.reshape(KD, KH * WQ * Ci, Wt * Co)


def _fused_conv_kernel(x_ref, scale_ref, shift_ref, w_ref, b_ref, *rest,
                       D, H, W, C, Wt, compute_stats, add_residual):
    i = 0
    res_ref = None
    if add_residual:
        res_ref = rest[i]; i += 1
    out_ref = rest[i]; i += 1
    sum_ref = sq_ref = None
    if compute_stats:
        sum_ref, sq_ref = rest[i], rest[i + 1]; i += 2
    lhs_ref, acc_ref = rest[i], rest[i + 1]

    Hp = H + 2
    T = W // Wt
    KB = (Wt + 2) * C

    @pl.when(pl.program_id(0) == 0)
    def _init():
        lhs_ref[...] = jnp.zeros_like(lhs_ref)
        if compute_stats:
            sum_ref[...] = jnp.zeros_like(sum_ref)
            sq_ref[...] = jnp.zeros_like(sq_ref)

    for d in range(D):
        xr = x_ref[0, d].astype(jnp.float32)
        xh = jnp.maximum(xr * scale_ref[...] + shift_ref[...], 0.0)
        xh = xh.astype(jnp.bfloat16)
        for t in range(T):
            w_lo = t * Wt - 1
            src_lo = max(w_lo, 0) * C
            src_hi = min(w_lo + Wt + 2, W) * C
            dst = (max(w_lo, 0) - w_lo) * C
            width = src_hi - src_lo
            xs = xh[:, src_lo:src_hi]
            for kh in range(3):
                r0 = (d + 1) * Hp + 1 - kh
                lhs_ref[t, r0:r0 + H, kh * KB + dst:kh * KB + dst + width] = xs

    for t in range(T):
        acc = None
        for kd in range(3):
            lhs = lhs_ref[t, kd * Hp:kd * Hp + D * Hp, :]
            mm = jnp.dot(lhs, w_ref[kd], preferred_element_type=jnp.float32)
            acc = mm if acc is None else acc + mm
        acc_ref[:, t * Wt * C:(t + 1) * Wt * C] = acc

    if compute_stats:
        s_acc = jnp.zeros_like(sum_ref)
        q_acc = jnp.zeros_like(sq_ref)
    for d in range(D):
        y = acc_ref[d * Hp:d * Hp + H, :] + b_ref[...]
        if compute_stats:
            s_acc += jnp.sum(y, axis=0, keepdims=True)
            q_acc += jnp.sum(y * y, axis=0, keepdims=True)
        if add_residual:
            y = y + res_ref[0, d].astype(jnp.float32)
        out_ref[0, d] = y.astype(out_ref.dtype)
    if compute_stats:
        sum_ref[...] += s_acc
        sq_ref[...] += q_acc


def _fused_conv(xin, scale, shift, w_dhwio, bias, *, C, out_dtype,
                residual=None, compute_stats=False):
    N, D, H, WC = xin.shape
    W = WC // C
    Wt = 8 if W % 8 == 0 else W
    T = W // Wt
    Hp = H + 2
    ROWS = (D + 2) * Hp + 2

    wexp = _expand_weight(w_dhwio, Wt).astype(jnp.bfloat16)
    scale_t = jnp.tile(scale.astype(jnp.float32), W).reshape(1, WC)
    shift_t = jnp.tile(shift.astype(jnp.float32), W).reshape(1, WC)
    bias_t = jnp.tile(bias.astype(jnp.float32), W).reshape(1, WC)

    body = functools.partial(
        _fused_conv_kernel, D=D, H=H, W=W, C=C, Wt=Wt,
        compute_stats=compute_stats, add_residual=residual is not None)

    in_specs = [
        pl.BlockSpec((1, D, H, WC), lambda n: (n, 0, 0, 0)),
        pl.BlockSpec((1, WC), lambda n: (0, 0)),
        pl.BlockSpec((1, WC), lambda n: (0, 0)),
        pl.BlockSpec(wexp.shape, lambda n: (0, 0, 0)),
        pl.BlockSpec((1, WC), lambda n: (0, 0)),
    ]
    args = [xin, scale_t, shift_t, wexp, bias_t]
    if residual is not None:
        in_specs.append(pl.BlockSpec((1, D, H, WC), lambda n: (n, 0, 0, 0)))
        args.append(residual)

    y_shape = jax.ShapeDtypeStruct((N, D, H, WC), out_dtype)
    y_spec = pl.BlockSpec((1, D, H, WC), lambda n: (n, 0, 0, 0))
    if compute_stats:
        stat_shape = jax.ShapeDtypeStruct((1, WC), jnp.float32)
        stat_spec = pl.BlockSpec((1, WC), lambda n: (0, 0))
        out_shape = (y_shape, stat_shape, stat_shape)
        out_specs = (y_spec, stat_spec, stat_spec)
    else:
        out_shape = y_shape
        out_specs = y_spec

    return pl.pallas_call(
        body,
        out_shape=out_shape,
        grid=(N,),
        in_specs=in_specs,
        out_specs=out_specs,
        scratch_shapes=[
            pltpu.VMEM((T, ROWS, 3 * (Wt + 2) * C), jnp.bfloat16),
            pltpu.VMEM((D * Hp, WC), jnp.float32),
        ],
        compiler_params=pltpu.CompilerParams(
            dimension_semantics=("arbitrary",),
            vmem_limit_bytes=_VMEM_LIMIT),
    )(*args)


def kernel(x, gamma1, beta1, w1, b1, gamma2, beta2, w2, b2):
    N, C, D, H, W = x.shape
    x_f = jnp.transpose(x, (0, 2, 3, 4, 1)).reshape(N, D, H, W * C)
    count = N * D * H * W

    s1, q1 = _channel_stats(x_f, C)
    scale1, shift1 = _bn_coeffs(s1, q1, count, gamma1, beta1)

    h1, hsum, hsq = _fused_conv(
        x_f, scale1, shift1, w1, b1, C=C, out_dtype=jnp.bfloat16,
        compute_stats=True)

    s2 = hsum.reshape(W, C).sum(axis=0)
    q2 = hsq.reshape(W, C).sum(axis=0)
    scale2, shift2 = _bn_coeffs(s2, q2, count, gamma2, beta2)

    out = _fused_conv(
        h1, scale2, shift2, w2, b2, C=C, out_dtype=x.dtype,
        residual=x_f, compute_stats=False)

    return out.reshape(N, D, H, W, C).transpose(0, 4, 1, 2, 3)
